# baseline (speedup 1.0000x reference)
import jax
import jax.numpy as jnp
from jax.experimental import pallas as pl
from jax.experimental.pallas import tpu as pltpu

_B, _N = 16, 8192
_IN, _H, _EMB, _HA, _OUT = 64, 128, 64, 128, 64
_NUM_AISLES = 32


def _lrelu(v):
    return jnp.maximum(v, v * jnp.asarray(0.01, v.dtype))


def _fused_kernel(x_ref, ids_ref, mask_ref,
                  w1_ref, b1_ref, w2_ref, b2_ref, w3_ref, b3_ref,
                  w4_ref, b4_ref, w5_ref, b5_ref, w6_ref, b6_ref,
                  out_ref):
    f32, bf = jnp.float32, jnp.bfloat16
    xb = x_ref[...].astype(bf)
    h = _lrelu(jnp.dot(xb, w1_ref[...], preferred_element_type=f32).astype(bf) + b1_ref[...])
    h = _lrelu(jnp.dot(h, w2_ref[...], preferred_element_type=f32).astype(bf) + b2_ref[...])
    zb = jnp.dot(h, w3_ref[...], preferred_element_type=f32).astype(bf) + b3_ref[...]

    ids = ids_ref[0]
    oh = (jnp.broadcast_to(ids, (_NUM_AISLES, _N)) ==
          jax.lax.broadcasted_iota(jnp.int32, (_NUM_AISLES, _N), 0)).astype(bf)
    sums = jax.lax.dot_general(oh, zb, (((1,), (0,)), ((), ())),
                               preferred_element_type=f32)
    counts = jnp.sum(oh.astype(f32), axis=1, keepdims=True)
    means = (sums / jnp.maximum(counts, 1.0)).astype(bf)
    g = jax.lax.dot_general(oh, means, (((0,), (0,)), ((), ())),
                            preferred_element_type=f32)

    cat = jnp.concatenate([zb, g.astype(bf)], axis=1)
    h2 = _lrelu(jnp.dot(cat, w4_ref[...], preferred_element_type=f32).astype(bf) + b4_ref[...])
    h2 = _lrelu(jnp.dot(h2, w5_ref[...], preferred_element_type=f32).astype(bf) + b5_ref[...])
    scores = jax.lax.dot_general(w6_ref[...], h2, (((0,), (1,)), ((), ())),
                                 preferred_element_type=f32) + b6_ref[0, 0]

    mk = mask_ref[0]
    logits = jnp.where(mk != 0, scores, -jnp.inf)
    mx = jnp.max(logits, axis=1, keepdims=True)
    e = jnp.exp(logits - mx)
    out_ref[0] = e / jnp.sum(e, axis=1, keepdims=True)


def kernel(x, aisle_nrs, mask, W1, b1, W2, b2, W3, b3, W4, b4, W5, b5, W6, b6):
    ids = aisle_nrs.astype(jnp.int32).reshape(_B, 1, _N)
    mask3 = mask.astype(jnp.int32).reshape(_B, 1, _N)
    bf = jnp.bfloat16

    full = lambda arr: pl.BlockSpec(arr.shape, lambda b: (0,) * arr.ndim)
    row2d = pl.BlockSpec((1, 1, _N), lambda b: (b, 0, 0))

    weights = [W1.astype(bf), b1.reshape(1, _H).astype(bf),
               W2.astype(bf), b2.reshape(1, _H).astype(bf),
               W3.astype(bf), b3.reshape(1, _EMB).astype(bf),
               W4.astype(bf), b4.reshape(1, _HA).astype(bf),
               W5.astype(bf), b5.reshape(1, _OUT).astype(bf),
               W6.astype(bf), b6.reshape(1, 1)]

    probs = pl.pallas_call(
        _fused_kernel,
        grid=(_B,),
        in_specs=[pl.BlockSpec((_N, _IN), lambda b: (b, 0)),
                  row2d, row2d] + [full(w) for w in weights],
        out_specs=pl.BlockSpec((1, 1, _N), lambda b: (b, 0, 0)),
        out_shape=jax.ShapeDtypeStruct((_B, 1, _N), jnp.float32),
        compiler_params=pltpu.CompilerParams(
            dimension_semantics=("parallel",)),
    )(x, ids, mask3, *weights)

    return probs.reshape(_B, _N)


# speedup vs baseline: 1.0523x; 1.0523x over previous
import jax
import jax.numpy as jnp
from jax.experimental import pallas as pl
from jax.experimental.pallas import tpu as pltpu

_B, _N = 16, 8192
_IN, _H, _EMB, _HA, _OUT = 64, 128, 64, 128, 64
_NUM_AISLES = 32


def _lrelu(v):
    return jnp.maximum(v, v * jnp.asarray(0.01, v.dtype))


def _fused_kernel(x_ref, ids_ref, mask_ref,
                  w1_ref, b1_ref, w2_ref, b2_ref, w3_ref, b3_ref,
                  w4_ref, b4_ref, w5_ref, b5_ref, w6_ref, b6_ref,
                  out_ref):
    f32, bf = jnp.float32, jnp.bfloat16
    w1 = w1_ref[...].astype(bf)
    w2 = w2_ref[...].astype(bf)
    w3 = w3_ref[...].astype(bf)
    w4 = w4_ref[...].astype(bf)
    w5 = w5_ref[...].astype(bf)
    w6 = w6_ref[...].astype(bf)
    xb = x_ref[...].astype(bf)
    h = _lrelu(jnp.dot(xb, w1, preferred_element_type=f32).astype(bf) + b1_ref[...].astype(bf))
    h = _lrelu(jnp.dot(h, w2, preferred_element_type=f32).astype(bf) + b2_ref[...].astype(bf))
    zb = jnp.dot(h, w3, preferred_element_type=f32).astype(bf) + b3_ref[...].astype(bf)

    ids = ids_ref[0]
    oh = (jnp.broadcast_to(ids, (_NUM_AISLES, _N)) ==
          jax.lax.broadcasted_iota(jnp.int32, (_NUM_AISLES, _N), 0)).astype(bf)
    sums = jax.lax.dot_general(oh, zb, (((1,), (0,)), ((), ())),
                               preferred_element_type=f32)
    counts = jnp.sum(oh.astype(f32), axis=1, keepdims=True)
    means = (sums / jnp.maximum(counts, 1.0)).astype(bf)
    g = jax.lax.dot_general(oh, means, (((0,), (0,)), ((), ())),
                            preferred_element_type=f32)

    cat = jnp.concatenate([zb, g.astype(bf)], axis=1)
    h2 = _lrelu(jnp.dot(cat, w4, preferred_element_type=f32).astype(bf) + b4_ref[...].astype(bf))
    h2 = _lrelu(jnp.dot(h2, w5, preferred_element_type=f32).astype(bf) + b5_ref[...].astype(bf))
    scores = jax.lax.dot_general(w6, h2, (((0,), (1,)), ((), ())),
                                 preferred_element_type=f32) + b6_ref[0, 0]

    mk = mask_ref[0]
    logits = jnp.where(mk != 0, scores, -jnp.inf)
    mx = jnp.max(logits, axis=1, keepdims=True)
    e = jnp.exp(logits - mx)
    out_ref[0] = e / jnp.sum(e, axis=1, keepdims=True)


def kernel(x, aisle_nrs, mask, W1, b1, W2, b2, W3, b3, W4, b4, W5, b5, W6, b6):
    ids = aisle_nrs.astype(jnp.int32).reshape(_B, 1, _N)
    mask3 = mask.astype(jnp.int32).reshape(_B, 1, _N)

    full = lambda arr: pl.BlockSpec(arr.shape, lambda b: (0,) * arr.ndim)
    row2d = pl.BlockSpec((1, 1, _N), lambda b: (b, 0, 0))

    weights = [W1, b1.reshape(1, _H), W2, b2.reshape(1, _H),
               W3, b3.reshape(1, _EMB), W4, b4.reshape(1, _HA),
               W5, b5.reshape(1, _OUT), W6, b6.reshape(1, 1)]

    probs = pl.pallas_call(
        _fused_kernel,
        grid=(_B,),
        in_specs=[pl.BlockSpec((_N, _IN), lambda b: (b, 0)),
                  row2d, row2d] + [full(w) for w in weights],
        out_specs=pl.BlockSpec((1, 1, _N), lambda b: (b, 0, 0)),
        out_shape=jax.ShapeDtypeStruct((_B, 1, _N), jnp.float32),
        compiler_params=pltpu.CompilerParams(
            dimension_semantics=("parallel",)),
    )(x, ids, mask3, *weights)

    return probs.reshape(_B, _N)


# speedup vs baseline: 1.0852x; 1.0312x over previous
import jax
import jax.numpy as jnp
from jax.experimental import pallas as pl
from jax.experimental.pallas import tpu as pltpu

_B, _N = 16, 8192
_IN, _H, _EMB, _HA, _OUT = 64, 128, 64, 128, 64
_NUM_AISLES = 32


def _lrelu(v):
    return jnp.maximum(v, v * jnp.asarray(0.01, v.dtype))


def _fused_kernel(x_ref, ids_ref, mask_ref,
                  w1_ref, b1_ref, w2_ref, b2_ref, w3_ref, b3_ref,
                  w4_ref, b4_ref, w5_ref, b5_ref, w6_ref, b6_ref,
                  out_ref):
    f32, bf = jnp.float32, jnp.bfloat16
    b = pl.program_id(0)
    w1 = w1_ref[...].astype(bf)
    w2 = w2_ref[...].astype(bf)
    w3 = w3_ref[...].astype(bf)
    w4 = w4_ref[...].astype(bf)
    w5 = w5_ref[...].astype(bf)
    w6 = w6_ref[...].astype(bf)
    xb = x_ref[...].astype(bf)
    h = _lrelu(jnp.dot(xb, w1, preferred_element_type=f32).astype(bf)
               + b1_ref[...].astype(bf)[None, :])
    h = _lrelu(jnp.dot(h, w2, preferred_element_type=f32).astype(bf)
               + b2_ref[...].astype(bf)[None, :])
    zb = (jnp.dot(h, w3, preferred_element_type=f32).astype(bf)
          + b3_ref[...].astype(bf)[None, :])

    ids = ids_ref[...][None, :]
    oh = (jnp.broadcast_to(ids, (_NUM_AISLES, _N)) ==
          jax.lax.broadcasted_iota(jnp.int32, (_NUM_AISLES, _N), 0)).astype(bf)
    sums = jax.lax.dot_general(oh, zb, (((1,), (0,)), ((), ())),
                               preferred_element_type=f32)
    counts = jnp.sum(oh.astype(f32), axis=1, keepdims=True)
    means = (sums / jnp.maximum(counts, 1.0)).astype(bf)
    g = jax.lax.dot_general(oh, means, (((0,), (0,)), ((), ())),
                            preferred_element_type=f32)

    cat = jnp.concatenate([zb, g.astype(bf)], axis=1)
    h2 = _lrelu(jnp.dot(cat, w4, preferred_element_type=f32).astype(bf)
                + b4_ref[...].astype(bf)[None, :])
    h2 = _lrelu(jnp.dot(h2, w5, preferred_element_type=f32).astype(bf)
                + b5_ref[...].astype(bf)[None, :])
    scores = jax.lax.dot_general(w6, h2, (((0,), (1,)), ((), ())),
                                 preferred_element_type=f32) + b6_ref[0]

    mk = mask_ref[pl.ds(b, 1), :]
    logits = jnp.where(mk != 0, scores, -jnp.inf)
    mx = jnp.max(logits, axis=1, keepdims=True)
    e = jnp.exp(logits - mx)
    out_ref[pl.ds(b, 1), :] = e / jnp.sum(e, axis=1, keepdims=True)


def kernel(x, aisle_nrs, mask, W1, b1, W2, b2, W3, b3, W4, b4, W5, b5, W6, b6):
    ids = aisle_nrs.astype(jnp.int32)

    full = lambda arr: pl.BlockSpec(arr.shape, lambda b: (0,) * arr.ndim)
    weights = [W1, b1, W2, b2, W3, b3, W4, b4, W5, b5, W6, b6]

    probs = pl.pallas_call(
        _fused_kernel,
        grid=(_B,),
        in_specs=[pl.BlockSpec((_N, _IN), lambda b: (b, 0)),
                  pl.BlockSpec((_N,), lambda b: (b,)),
                  full(mask)] + [full(w) for w in weights],
        out_specs=pl.BlockSpec((_B, _N), lambda b: (0, 0)),
        out_shape=jax.ShapeDtypeStruct((_B, _N), jnp.float32),
        compiler_params=pltpu.CompilerParams(
            dimension_semantics=("arbitrary",)),
    )(x, ids, mask, *weights)

    return probs
